# baseline (speedup 1.0000x reference)
import functools
import math

import jax
import jax.numpy as jnp
import numpy as np
from jax import lax
from jax.experimental import pallas as pl
from jax.experimental.pallas import tpu as pltpu
from jax.experimental.pallas import tpu_sc as plsc

_EPS = 1e-8
_B = 512
_DIM = 128
_LOG_LO = float(math.log(1e-10))
_LOG_HI = float(math.log(1e4))

_rng = np.random.default_rng(0)
_SAMP = [_rng.integers(0, 20000, size=_B).astype(np.int32) for _ in range(6)]
del _rng

_NC = 2
_NS = 16
_NW = _NC * _NS

_CLS_A = 5 * _B
_CLS_B = 8 * _B
_REL_N = 3 * _B
_APW = _CLS_A // _NW
_BPWC = _CLS_B // _NW
_RPW = _REL_N // _NW


@functools.cache
def _build_sc_gather_a():
    return functools.partial(
        pl.kernel,
        mesh=plsc.VectorSubcoreMesh(core_axis_name="c", subcore_axis_name="s"),
        out_type=[
            jax.ShapeDtypeStruct((_CLS_A, _DIM), jnp.float32),
            jax.ShapeDtypeStruct((_CLS_A, _DIM), jnp.float32),
        ],
        scratch_types=[
            pltpu.VMEM((_APW,), jnp.int32),
            pltpu.VMEM((_APW, _DIM), jnp.float32),
            pltpu.VMEM((_APW, _DIM), jnp.float32),
            pltpu.SemaphoreType.DMA,
        ],
    )(_sc_gather_a_body)


def _sc_gather_a_body(min_hbm, del_hbm, idx_hbm, out_min, out_del,
                      idx0, rm0, rd0, sem):
    wid = lax.axis_index("s") * _NC + lax.axis_index("c")
    base = wid * _APW
    pltpu.sync_copy(idx_hbm.at[pl.ds(base, _APW)], idx0)
    cps = [
        pltpu.async_copy(min_hbm.at[idx0], rm0, sem),
        pltpu.async_copy(del_hbm.at[idx0], rd0, sem),
    ]
    for c in cps:
        c.wait()
    pltpu.sync_copy(rm0, out_min.at[pl.ds(base, _APW)])
    pltpu.sync_copy(rd0, out_del.at[pl.ds(base, _APW)])


@functools.cache
def _build_sc_gather_b():
    return functools.partial(
        pl.kernel,
        mesh=plsc.VectorSubcoreMesh(core_axis_name="c", subcore_axis_name="s"),
        out_type=[
            jax.ShapeDtypeStruct((_CLS_B, _DIM), jnp.float32),
            jax.ShapeDtypeStruct((_CLS_B, _DIM), jnp.float32),
            jax.ShapeDtypeStruct((_REL_N, _DIM), jnp.float32),
            jax.ShapeDtypeStruct((_REL_N, _DIM), jnp.float32),
        ],
        scratch_types=[
            pltpu.VMEM((_BPWC,), jnp.int32),
            pltpu.VMEM((_RPW,), jnp.int32),
            pltpu.VMEM((_BPWC, _DIM), jnp.float32),
            pltpu.VMEM((_BPWC, _DIM), jnp.float32),
            pltpu.VMEM((_RPW, _DIM), jnp.float32),
            pltpu.VMEM((_RPW, _DIM), jnp.float32),
            pltpu.SemaphoreType.DMA,
        ],
    )(_sc_gather_b_body)


def _sc_gather_b_body(min_hbm, del_hbm, rel_hbm, scal_hbm, idx_hbm,
                      out_min, out_del, out_rel, out_scal,
                      idx0, idxr, rm0, rd0, rr, rs, sem):
    wid = lax.axis_index("s") * _NC + lax.axis_index("c")
    base = wid * _BPWC
    rbase = _CLS_B + wid * _RPW
    pltpu.sync_copy(idx_hbm.at[pl.ds(base, _BPWC)], idx0)
    pltpu.sync_copy(idx_hbm.at[pl.ds(rbase, _RPW)], idxr)
    cps = [
        pltpu.async_copy(min_hbm.at[idx0], rm0, sem),
        pltpu.async_copy(del_hbm.at[idx0], rd0, sem),
        pltpu.async_copy(rel_hbm.at[idxr], rr, sem),
        pltpu.async_copy(scal_hbm.at[idxr], rs, sem),
    ]
    for c in cps:
        c.wait()
    pltpu.sync_copy(rm0, out_min.at[pl.ds(base, _BPWC)])
    pltpu.sync_copy(rd0, out_del.at[pl.ds(base, _BPWC)])
    pltpu.sync_copy(rr, out_rel.at[pl.ds(wid * _RPW, _RPW)])
    pltpu.sync_copy(rs, out_scal.at[pl.ds(wid * _RPW, _RPW)])


def _softplus(x):
    return jnp.maximum(x, 0.0) + jnp.log1p(jnp.exp(-jnp.abs(x)))


def _tc_body(gmin_a_ref, gdel_a_ref, gmin_b_ref, gdel_b_ref,
             grel_ref, gscal_ref, out_ref):
    mn_a = gmin_a_ref[...]
    mx_a = mn_a + jnp.exp(gdel_a_ref[...])
    mn_b = gmin_b_ref[...]
    mx_b = mn_b + jnp.exp(gdel_b_ref[...])
    rel_all = grel_ref[...]
    scal_all = gscal_ref[...]

    def seg(i):
        if i < 5:
            return mn_a[i * _B:(i + 1) * _B], mx_a[i * _B:(i + 1) * _B]
        i -= 5
        return mn_b[i * _B:(i + 1) * _B], mx_b[i * _B:(i + 1) * _B]

    def rseg(i):
        return rel_all[i * _B:(i + 1) * _B], scal_all[i * _B:(i + 1) * _B]

    def logvol(mn, mx):
        sp = _softplus(mx - mn)
        return jnp.clip(jnp.sum(jnp.log(sp), axis=1, keepdims=True),
                        _LOG_LO, _LOG_HI)

    def inclusion(mn1, mx1, mn2, mx2):
        imn = jnp.maximum(mn1, mn2)
        imx = jnp.minimum(mx1, mx2)
        return 1.0 - jnp.exp(logvol(imn, imx) - logvol(mn1, mx1))

    def reg(mn, mx):
        d = mx - mn
        t = jnp.maximum(mn + d - 1.0 + _EPS, 0.0)
        nrm = jnp.sqrt(jnp.sum(mn * mn))
        return jnp.sum(t) * (1.0 / (_B * _DIM)) + jnp.maximum(nrm - 1.0, 0.0)

    amn, amx = seg(0)
    bmn, bmx = seg(1)
    total = jnp.sum(inclusion(amn, amx, bmn, bmx)) + reg(amn, amx) + reg(bmn, bmx)

    amn, amx = seg(2)
    bmn, bmx = seg(3)
    cmn, cmx = seg(4)
    imn = jnp.maximum(amn, bmn)
    imx = jnp.minimum(amx, bmx)
    total += (jnp.sum(inclusion(imn, imx, cmn, cmx))
              + reg(imn, imx) + reg(amn, amx) + reg(bmn, bmx) + reg(cmn, cmx))

    amn, amx = seg(5)
    bmn, bmx = seg(6)
    rel, sc = rseg(0)
    s = sc + _EPS
    tmn = amn * s + rel
    tmx = amx * s + rel
    total += (jnp.sum(inclusion(tmn, tmx, bmn, bmx))
              + reg(tmn, tmx) + reg(amn, amx) + reg(bmn, bmx))

    amn, amx = seg(7)
    bmn, bmx = seg(8)
    rel, sc = rseg(1)
    s = sc + _EPS
    tmn = (amn - rel) / s
    tmx = (amx - rel) / s
    total += (jnp.sum(inclusion(tmn, tmx, bmn, bmx))
              + reg(tmn, tmx) + reg(amn, amx) + reg(bmn, bmx))

    amn, amx = seg(9)
    bmn, bmx = seg(10)
    imn = jnp.maximum(amn, bmn)
    imx = jnp.minimum(amx, bmx)
    dis = jnp.exp(logvol(imn, imx) - (logvol(amn, amx) + logvol(bmn, bmx)))
    total += jnp.sum(dis) + reg(amn, amx) + reg(bmn, bmx)

    amn, amx = seg(11)
    bmn, bmx = seg(12)
    rel, sc = rseg(2)
    s = sc + _EPS
    tmn = amn * s + rel
    tmx = amx * s + rel
    imn = jnp.maximum(tmn, bmn)
    imx = jnp.minimum(tmx, bmx)
    neg = jnp.exp(logvol(imn, imx) - logvol(tmn, tmx))
    total += jnp.sum(neg) + reg(tmn, tmx) + reg(amn, amx) + reg(bmn, bmx)

    out_ref[0, 0] = total


def _tc_loss(gmin_a, gdel_a, gmin_b, gdel_b, grel, gscal):
    return pl.pallas_call(
        _tc_body,
        out_shape=jax.ShapeDtypeStruct((1, 1), jnp.float32),
        out_specs=pl.BlockSpec(memory_space=pltpu.SMEM),
    )(gmin_a, gdel_a, gmin_b, gdel_b, grel, gscal)


def kernel(nf1, nf2, nf3, nf4, disjoint, nf3_neg0, min_embedding,
           delta_embedding, relation_embedding, scaling_embedding):
    s1, s2, s3, s4, s5, s6 = _SAMP
    nf1_s = nf1[s1]
    nf2_s = nf2[s2]
    nf3_s = nf3[s3]
    nf4_s = nf4[s4]
    dis_s = disjoint[s5]
    neg_s = nf3_neg0[s6]

    idx_a = jnp.concatenate([
        nf1_s[:, 0], nf1_s[:, 1],
        nf2_s[:, 0], nf2_s[:, 1], nf2_s[:, 2],
    ]).astype(jnp.int32)
    idx_b = jnp.concatenate([
        nf3_s[:, 0], nf3_s[:, 2],
        nf4_s[:, 1], nf4_s[:, 2],
        dis_s[:, 0], dis_s[:, 1],
        neg_s[:, 0], neg_s[:, 2],
        nf3_s[:, 1], nf4_s[:, 0], neg_s[:, 1],
    ]).astype(jnp.int32)

    gmin_a, gdel_a = _build_sc_gather_a()(
        min_embedding, delta_embedding, idx_a)
    gmin_b, gdel_b, grel, gscal = _build_sc_gather_b()(
        min_embedding, delta_embedding, relation_embedding, scaling_embedding,
        idx_b)
    res = _tc_loss(gmin_a, gdel_a, gmin_b, gdel_b, grel, gscal)
    return res[0, 0]


# speedup vs baseline: 1.0236x; 1.0236x over previous
import functools
import math

import jax
import jax.numpy as jnp
import numpy as np
from jax import lax
from jax.experimental import pallas as pl
from jax.experimental.pallas import tpu as pltpu
from jax.experimental.pallas import tpu_sc as plsc

_EPS = 1e-8
_B = 512
_DIM = 128
_LOG_LO = float(math.log(1e-10))
_LOG_HI = float(math.log(1e4))

_rng = np.random.default_rng(0)
_SAMP = [_rng.integers(0, 20000, size=_B).astype(np.int32) for _ in range(6)]
del _rng

_NC = 2
_NS = 16
_NW = _NC * _NS

_CLS_A = 9 * _B
_REL_A = 2 * _B
_CLS_B = 4 * _B
_REL_B = 1 * _B
_APW = _CLS_A // _NW
_ACH = _APW // 2
_ARPW = _REL_A // _NW
_BPWC = _CLS_B // _NW
_BRPW = _REL_B // _NW


@functools.cache
def _build_sc_gather_a():
    return functools.partial(
        pl.kernel,
        mesh=plsc.VectorSubcoreMesh(core_axis_name="c", subcore_axis_name="s"),
        out_type=[
            jax.ShapeDtypeStruct((_CLS_A, _DIM), jnp.float32),
            jax.ShapeDtypeStruct((_CLS_A, _DIM), jnp.float32),
            jax.ShapeDtypeStruct((_REL_A, _DIM), jnp.float32),
            jax.ShapeDtypeStruct((_REL_A, _DIM), jnp.float32),
        ],
        scratch_types=[
            pltpu.VMEM((_ACH,), jnp.int32),
            pltpu.VMEM((_ACH,), jnp.int32),
            pltpu.VMEM((_ARPW,), jnp.int32),
            pltpu.VMEM((_ACH, _DIM), jnp.float32),
            pltpu.VMEM((_ACH, _DIM), jnp.float32),
            pltpu.VMEM((_ACH, _DIM), jnp.float32),
            pltpu.VMEM((_ACH, _DIM), jnp.float32),
            pltpu.VMEM((_ARPW, _DIM), jnp.float32),
            pltpu.VMEM((_ARPW, _DIM), jnp.float32),
            pltpu.SemaphoreType.DMA,
        ],
    )(_sc_gather_a_body)


def _sc_gather_a_body(min_hbm, del_hbm, rel_hbm, scal_hbm, idx_hbm,
                      out_min, out_del, out_rel, out_scal,
                      idx0, idx1, idxr, rm0, rd0, rm1, rd1, rr, rs, sem):
    wid = lax.axis_index("s") * _NC + lax.axis_index("c")
    base = wid * _APW
    rbase = _CLS_A + wid * _ARPW
    pltpu.sync_copy(idx_hbm.at[pl.ds(base, _ACH)], idx0)
    pltpu.sync_copy(idx_hbm.at[pl.ds(base + _ACH, _ACH)], idx1)
    pltpu.sync_copy(idx_hbm.at[pl.ds(rbase, _ARPW)], idxr)
    cps = [
        pltpu.async_copy(min_hbm.at[idx0], rm0, sem),
        pltpu.async_copy(del_hbm.at[idx0], rd0, sem),
        pltpu.async_copy(min_hbm.at[idx1], rm1, sem),
        pltpu.async_copy(del_hbm.at[idx1], rd1, sem),
        pltpu.async_copy(rel_hbm.at[idxr], rr, sem),
        pltpu.async_copy(scal_hbm.at[idxr], rs, sem),
    ]
    for c in cps:
        c.wait()
    pltpu.sync_copy(rm0, out_min.at[pl.ds(base, _ACH)])
    pltpu.sync_copy(rd0, out_del.at[pl.ds(base, _ACH)])
    pltpu.sync_copy(rm1, out_min.at[pl.ds(base + _ACH, _ACH)])
    pltpu.sync_copy(rd1, out_del.at[pl.ds(base + _ACH, _ACH)])
    pltpu.sync_copy(rr, out_rel.at[pl.ds(wid * _ARPW, _ARPW)])
    pltpu.sync_copy(rs, out_scal.at[pl.ds(wid * _ARPW, _ARPW)])


@functools.cache
def _build_sc_gather_b():
    return functools.partial(
        pl.kernel,
        mesh=plsc.VectorSubcoreMesh(core_axis_name="c", subcore_axis_name="s"),
        out_type=[
            jax.ShapeDtypeStruct((_CLS_B, _DIM), jnp.float32),
            jax.ShapeDtypeStruct((_CLS_B, _DIM), jnp.float32),
            jax.ShapeDtypeStruct((_REL_B, _DIM), jnp.float32),
            jax.ShapeDtypeStruct((_REL_B, _DIM), jnp.float32),
        ],
        scratch_types=[
            pltpu.VMEM((_BPWC,), jnp.int32),
            pltpu.VMEM((_BRPW,), jnp.int32),
            pltpu.VMEM((_BPWC, _DIM), jnp.float32),
            pltpu.VMEM((_BPWC, _DIM), jnp.float32),
            pltpu.VMEM((_BRPW, _DIM), jnp.float32),
            pltpu.VMEM((_BRPW, _DIM), jnp.float32),
            pltpu.SemaphoreType.DMA,
        ],
    )(_sc_gather_b_body)


def _sc_gather_b_body(min_hbm, del_hbm, rel_hbm, scal_hbm, idx_hbm,
                      out_min, out_del, out_rel, out_scal,
                      idx0, idxr, rm0, rd0, rr, rs, sem):
    wid = lax.axis_index("s") * _NC + lax.axis_index("c")
    base = wid * _BPWC
    rbase = _CLS_B + wid * _BRPW
    pltpu.sync_copy(idx_hbm.at[pl.ds(base, _BPWC)], idx0)
    pltpu.sync_copy(idx_hbm.at[pl.ds(rbase, _BRPW)], idxr)
    cps = [
        pltpu.async_copy(min_hbm.at[idx0], rm0, sem),
        pltpu.async_copy(del_hbm.at[idx0], rd0, sem),
        pltpu.async_copy(rel_hbm.at[idxr], rr, sem),
        pltpu.async_copy(scal_hbm.at[idxr], rs, sem),
    ]
    for c in cps:
        c.wait()
    pltpu.sync_copy(rm0, out_min.at[pl.ds(base, _BPWC)])
    pltpu.sync_copy(rd0, out_del.at[pl.ds(base, _BPWC)])
    pltpu.sync_copy(rr, out_rel.at[pl.ds(wid * _BRPW, _BRPW)])
    pltpu.sync_copy(rs, out_scal.at[pl.ds(wid * _BRPW, _BRPW)])


def _softplus(x):
    return jnp.maximum(x, 0.0) + jnp.log1p(jnp.exp(-jnp.abs(x)))


def _tc_body(gmin_a_ref, gdel_a_ref, gmin_b_ref, gdel_b_ref,
             grel_a_ref, gscal_a_ref, grel_b_ref, gscal_b_ref, out_ref):
    mn_a = gmin_a_ref[...]
    mx_a = mn_a + jnp.exp(gdel_a_ref[...])
    mn_b = gmin_b_ref[...]
    mx_b = mn_b + jnp.exp(gdel_b_ref[...])
    rel_a = grel_a_ref[...]
    scal_a = gscal_a_ref[...]
    rel_b = grel_b_ref[...]
    scal_b = gscal_b_ref[...]

    def seg(i):
        if i < 9:
            return mn_a[i * _B:(i + 1) * _B], mx_a[i * _B:(i + 1) * _B]
        i -= 9
        return mn_b[i * _B:(i + 1) * _B], mx_b[i * _B:(i + 1) * _B]

    def rseg(i):
        if i < 2:
            return rel_a[i * _B:(i + 1) * _B], scal_a[i * _B:(i + 1) * _B]
        return rel_b, scal_b

    def logvol(mn, mx):
        sp = _softplus(mx - mn)
        return jnp.clip(jnp.sum(jnp.log(sp), axis=1, keepdims=True),
                        _LOG_LO, _LOG_HI)

    def inclusion(mn1, mx1, mn2, mx2):
        imn = jnp.maximum(mn1, mn2)
        imx = jnp.minimum(mx1, mx2)
        return 1.0 - jnp.exp(logvol(imn, imx) - logvol(mn1, mx1))

    def reg(mn, mx):
        d = mx - mn
        t = jnp.maximum(mn + d - 1.0 + _EPS, 0.0)
        nrm = jnp.sqrt(jnp.sum(mn * mn))
        return jnp.sum(t) * (1.0 / (_B * _DIM)) + jnp.maximum(nrm - 1.0, 0.0)

    amn, amx = seg(0)
    bmn, bmx = seg(1)
    total = jnp.sum(inclusion(amn, amx, bmn, bmx)) + reg(amn, amx) + reg(bmn, bmx)

    amn, amx = seg(2)
    bmn, bmx = seg(3)
    cmn, cmx = seg(4)
    imn = jnp.maximum(amn, bmn)
    imx = jnp.minimum(amx, bmx)
    total += (jnp.sum(inclusion(imn, imx, cmn, cmx))
              + reg(imn, imx) + reg(amn, amx) + reg(bmn, bmx) + reg(cmn, cmx))

    amn, amx = seg(5)
    bmn, bmx = seg(6)
    rel, sc = rseg(0)
    s = sc + _EPS
    tmn = amn * s + rel
    tmx = amx * s + rel
    total += (jnp.sum(inclusion(tmn, tmx, bmn, bmx))
              + reg(tmn, tmx) + reg(amn, amx) + reg(bmn, bmx))

    amn, amx = seg(7)
    bmn, bmx = seg(8)
    rel, sc = rseg(1)
    s = sc + _EPS
    tmn = (amn - rel) / s
    tmx = (amx - rel) / s
    total += (jnp.sum(inclusion(tmn, tmx, bmn, bmx))
              + reg(tmn, tmx) + reg(amn, amx) + reg(bmn, bmx))

    amn, amx = seg(9)
    bmn, bmx = seg(10)
    imn = jnp.maximum(amn, bmn)
    imx = jnp.minimum(amx, bmx)
    dis = jnp.exp(logvol(imn, imx) - (logvol(amn, amx) + logvol(bmn, bmx)))
    total += jnp.sum(dis) + reg(amn, amx) + reg(bmn, bmx)

    amn, amx = seg(11)
    bmn, bmx = seg(12)
    rel, sc = rseg(2)
    s = sc + _EPS
    tmn = amn * s + rel
    tmx = amx * s + rel
    imn = jnp.maximum(tmn, bmn)
    imx = jnp.minimum(tmx, bmx)
    neg = jnp.exp(logvol(imn, imx) - logvol(tmn, tmx))
    total += jnp.sum(neg) + reg(tmn, tmx) + reg(amn, amx) + reg(bmn, bmx)

    out_ref[0, 0] = total


def _tc_loss(gmin_a, gdel_a, gmin_b, gdel_b, grel_a, gscal_a, grel_b, gscal_b):
    return pl.pallas_call(
        _tc_body,
        out_shape=jax.ShapeDtypeStruct((1, 1), jnp.float32),
        out_specs=pl.BlockSpec(memory_space=pltpu.SMEM),
    )(gmin_a, gdel_a, gmin_b, gdel_b, grel_a, gscal_a, grel_b, gscal_b)


def kernel(nf1, nf2, nf3, nf4, disjoint, nf3_neg0, min_embedding,
           delta_embedding, relation_embedding, scaling_embedding):
    s1, s2, s3, s4, s5, s6 = _SAMP
    nf1_s = nf1[s1]
    nf2_s = nf2[s2]
    nf3_s = nf3[s3]
    nf4_s = nf4[s4]
    dis_s = disjoint[s5]
    neg_s = nf3_neg0[s6]

    idx_a = jnp.concatenate([
        nf1_s[:, 0], nf1_s[:, 1],
        nf2_s[:, 0], nf2_s[:, 1], nf2_s[:, 2],
        nf3_s[:, 0], nf3_s[:, 2],
        nf4_s[:, 1], nf4_s[:, 2],
        nf3_s[:, 1], nf4_s[:, 0],
    ]).astype(jnp.int32)
    idx_b = jnp.concatenate([
        dis_s[:, 0], dis_s[:, 1],
        neg_s[:, 0], neg_s[:, 2],
        neg_s[:, 1],
    ]).astype(jnp.int32)

    gmin_a, gdel_a, grel_a, gscal_a = _build_sc_gather_a()(
        min_embedding, delta_embedding, relation_embedding, scaling_embedding,
        idx_a)
    gmin_b, gdel_b, grel_b, gscal_b = _build_sc_gather_b()(
        min_embedding, delta_embedding, relation_embedding, scaling_embedding,
        idx_b)
    res = _tc_loss(gmin_a, gdel_a, gmin_b, gdel_b,
                   grel_a, gscal_a, grel_b, gscal_b)
    return res[0, 0]


# speedup vs baseline: 1.0507x; 1.0265x over previous
import functools
import math

import jax
import jax.numpy as jnp
import numpy as np
from jax import lax
from jax.experimental import pallas as pl
from jax.experimental.pallas import tpu as pltpu
from jax.experimental.pallas import tpu_sc as plsc

_EPS = 1e-8
_B = 512
_DIM = 128
_LOG_LO = float(math.log(1e-10))
_LOG_HI = float(math.log(1e4))

_rng = np.random.default_rng(0)
_SAMP = [_rng.integers(0, 20000, size=_B).astype(np.int32) for _ in range(6)]
del _rng

_NC = 2
_NS = 16
_NW = _NC * _NS

_CLS_A = 9 * _B
_REL_A = 2 * _B
_CLS_B = 4 * _B
_REL_B = 1 * _B
_APW = _CLS_A // _NW
_ACH = _APW // 2
_ARPW = _REL_A // _NW
_BPWC = _CLS_B // _NW
_BRPW = _REL_B // _NW


@functools.cache
def _build_sc_gather_a():
    return functools.partial(
        pl.kernel,
        mesh=plsc.VectorSubcoreMesh(core_axis_name="c", subcore_axis_name="s"),
        out_type=[
            jax.ShapeDtypeStruct((_CLS_A, _DIM), jnp.float32),
            jax.ShapeDtypeStruct((_CLS_A, _DIM), jnp.float32),
            jax.ShapeDtypeStruct((_REL_A, _DIM), jnp.float32),
            jax.ShapeDtypeStruct((_REL_A, _DIM), jnp.float32),
        ],
        scratch_types=[
            pltpu.VMEM((_ACH,), jnp.int32),
            pltpu.VMEM((_ACH,), jnp.int32),
            pltpu.VMEM((_ARPW,), jnp.int32),
            pltpu.VMEM((_ACH, _DIM), jnp.float32),
            pltpu.VMEM((_ACH, _DIM), jnp.float32),
            pltpu.VMEM((_ACH, _DIM), jnp.float32),
            pltpu.VMEM((_ACH, _DIM), jnp.float32),
            pltpu.VMEM((_ARPW, _DIM), jnp.float32),
            pltpu.VMEM((_ARPW, _DIM), jnp.float32),
            pltpu.SemaphoreType.DMA,
        ],
    )(_sc_gather_a_body)


def _sc_gather_a_body(min_hbm, del_hbm, rel_hbm, scal_hbm, idx_hbm,
                      out_min, out_del, out_rel, out_scal,
                      idx0, idx1, idxr, rm0, rd0, rm1, rd1, rr, rs, sem):
    wid = lax.axis_index("s") * _NC + lax.axis_index("c")
    base = wid * _APW
    rbase = _CLS_A + wid * _ARPW
    pltpu.sync_copy(idx_hbm.at[pl.ds(base, _ACH)], idx0)
    pltpu.sync_copy(idx_hbm.at[pl.ds(base + _ACH, _ACH)], idx1)
    pltpu.sync_copy(idx_hbm.at[pl.ds(rbase, _ARPW)], idxr)
    cps = [
        pltpu.async_copy(min_hbm.at[idx0], rm0, sem),
        pltpu.async_copy(del_hbm.at[idx0], rd0, sem),
        pltpu.async_copy(min_hbm.at[idx1], rm1, sem),
        pltpu.async_copy(del_hbm.at[idx1], rd1, sem),
        pltpu.async_copy(rel_hbm.at[idxr], rr, sem),
        pltpu.async_copy(scal_hbm.at[idxr], rs, sem),
    ]
    for c in cps:
        c.wait()
    pltpu.sync_copy(rm0, out_min.at[pl.ds(base, _ACH)])
    pltpu.sync_copy(rd0, out_del.at[pl.ds(base, _ACH)])
    pltpu.sync_copy(rm1, out_min.at[pl.ds(base + _ACH, _ACH)])
    pltpu.sync_copy(rd1, out_del.at[pl.ds(base + _ACH, _ACH)])
    pltpu.sync_copy(rr, out_rel.at[pl.ds(wid * _ARPW, _ARPW)])
    pltpu.sync_copy(rs, out_scal.at[pl.ds(wid * _ARPW, _ARPW)])


@functools.cache
def _build_sc_gather_b():
    return functools.partial(
        pl.kernel,
        mesh=plsc.VectorSubcoreMesh(core_axis_name="c", subcore_axis_name="s"),
        out_type=[
            jax.ShapeDtypeStruct((_CLS_B, _DIM), jnp.float32),
            jax.ShapeDtypeStruct((_CLS_B, _DIM), jnp.float32),
            jax.ShapeDtypeStruct((_REL_B, _DIM), jnp.float32),
            jax.ShapeDtypeStruct((_REL_B, _DIM), jnp.float32),
        ],
        scratch_types=[
            pltpu.VMEM((_BPWC,), jnp.int32),
            pltpu.VMEM((_BRPW,), jnp.int32),
            pltpu.VMEM((_BPWC, _DIM), jnp.float32),
            pltpu.VMEM((_BPWC, _DIM), jnp.float32),
            pltpu.VMEM((_BRPW, _DIM), jnp.float32),
            pltpu.VMEM((_BRPW, _DIM), jnp.float32),
            pltpu.SemaphoreType.DMA,
        ],
    )(_sc_gather_b_body)


def _sc_gather_b_body(min_hbm, del_hbm, rel_hbm, scal_hbm, idx_hbm,
                      out_min, out_del, out_rel, out_scal,
                      idx0, idxr, rm0, rd0, rr, rs, sem):
    wid = lax.axis_index("s") * _NC + lax.axis_index("c")
    base = wid * _BPWC
    rbase = _CLS_B + wid * _BRPW
    pltpu.sync_copy(idx_hbm.at[pl.ds(base, _BPWC)], idx0)
    pltpu.sync_copy(idx_hbm.at[pl.ds(rbase, _BRPW)], idxr)
    cps = [
        pltpu.async_copy(min_hbm.at[idx0], rm0, sem),
        pltpu.async_copy(del_hbm.at[idx0], rd0, sem),
        pltpu.async_copy(rel_hbm.at[idxr], rr, sem),
        pltpu.async_copy(scal_hbm.at[idxr], rs, sem),
    ]
    for c in cps:
        c.wait()
    pltpu.sync_copy(rm0, out_min.at[pl.ds(base, _BPWC)])
    pltpu.sync_copy(rd0, out_del.at[pl.ds(base, _BPWC)])
    pltpu.sync_copy(rr, out_rel.at[pl.ds(wid * _BRPW, _BRPW)])
    pltpu.sync_copy(rs, out_scal.at[pl.ds(wid * _BRPW, _BRPW)])


def _softplus(x):
    return jnp.maximum(x, 0.0) + jnp.log1p(jnp.exp(-jnp.abs(x)))


def _logvol(mn, mx):
    sp = _softplus(mx - mn)
    return jnp.clip(jnp.sum(jnp.log(sp), axis=1, keepdims=True),
                    _LOG_LO, _LOG_HI)


def _inclusion(mn1, mx1, mn2, mx2):
    imn = jnp.maximum(mn1, mn2)
    imx = jnp.minimum(mx1, mx2)
    return 1.0 - jnp.exp(_logvol(imn, imx) - _logvol(mn1, mx1))


def _reg(mn, mx):
    d = mx - mn
    t = jnp.maximum(mn + d - 1.0 + _EPS, 0.0)
    nrm = jnp.sqrt(jnp.sum(mn * mn))
    return jnp.sum(t) * (1.0 / (_B * _DIM)) + jnp.maximum(nrm - 1.0, 0.0)


def _tc_a_body(gmin_a_ref, gdel_a_ref, grel_a_ref, gscal_a_ref, out_ref):
    mn_a = gmin_a_ref[...]
    mx_a = mn_a + jnp.exp(gdel_a_ref[...])
    rel_a = grel_a_ref[...]
    scal_a = gscal_a_ref[...]

    def seg(i):
        return mn_a[i * _B:(i + 1) * _B], mx_a[i * _B:(i + 1) * _B]

    def rseg(i):
        return rel_a[i * _B:(i + 1) * _B], scal_a[i * _B:(i + 1) * _B]

    amn, amx = seg(0)
    bmn, bmx = seg(1)
    total = jnp.sum(_inclusion(amn, amx, bmn, bmx)) + _reg(amn, amx) + _reg(bmn, bmx)

    amn, amx = seg(2)
    bmn, bmx = seg(3)
    cmn, cmx = seg(4)
    imn = jnp.maximum(amn, bmn)
    imx = jnp.minimum(amx, bmx)
    total += (jnp.sum(_inclusion(imn, imx, cmn, cmx))
              + _reg(imn, imx) + _reg(amn, amx) + _reg(bmn, bmx) + _reg(cmn, cmx))

    amn, amx = seg(5)
    bmn, bmx = seg(6)
    rel, sc = rseg(0)
    s = sc + _EPS
    tmn = amn * s + rel
    tmx = amx * s + rel
    total += (jnp.sum(_inclusion(tmn, tmx, bmn, bmx))
              + _reg(tmn, tmx) + _reg(amn, amx) + _reg(bmn, bmx))

    amn, amx = seg(7)
    bmn, bmx = seg(8)
    rel, sc = rseg(1)
    s = sc + _EPS
    tmn = (amn - rel) / s
    tmx = (amx - rel) / s
    total += (jnp.sum(_inclusion(tmn, tmx, bmn, bmx))
              + _reg(tmn, tmx) + _reg(amn, amx) + _reg(bmn, bmx))

    out_ref[0, 0] = total


def _tc_b_body(gmin_b_ref, gdel_b_ref, grel_b_ref, gscal_b_ref,
               partial_ref, out_ref):
    mn_b = gmin_b_ref[...]
    mx_b = mn_b + jnp.exp(gdel_b_ref[...])
    rel_b = grel_b_ref[...]
    scal_b = gscal_b_ref[...]

    def seg(i):
        return mn_b[i * _B:(i + 1) * _B], mx_b[i * _B:(i + 1) * _B]

    amn, amx = seg(0)
    bmn, bmx = seg(1)
    imn = jnp.maximum(amn, bmn)
    imx = jnp.minimum(amx, bmx)
    dis = jnp.exp(_logvol(imn, imx) - (_logvol(amn, amx) + _logvol(bmn, bmx)))
    total = jnp.sum(dis) + _reg(amn, amx) + _reg(bmn, bmx)

    amn, amx = seg(2)
    bmn, bmx = seg(3)
    s = scal_b + _EPS
    tmn = amn * s + rel_b
    tmx = amx * s + rel_b
    imn = jnp.maximum(tmn, bmn)
    imx = jnp.minimum(tmx, bmx)
    neg = jnp.exp(_logvol(imn, imx) - _logvol(tmn, tmx))
    total += jnp.sum(neg) + _reg(tmn, tmx) + _reg(amn, amx) + _reg(bmn, bmx)

    out_ref[0, 0] = total + partial_ref[0, 0]


def _tc_loss_a(gmin_a, gdel_a, grel_a, gscal_a):
    return pl.pallas_call(
        _tc_a_body,
        out_shape=jax.ShapeDtypeStruct((1, 1), jnp.float32),
        out_specs=pl.BlockSpec(memory_space=pltpu.SMEM),
    )(gmin_a, gdel_a, grel_a, gscal_a)


def _tc_loss_b(gmin_b, gdel_b, grel_b, gscal_b, partial):
    return pl.pallas_call(
        _tc_b_body,
        out_shape=jax.ShapeDtypeStruct((1, 1), jnp.float32),
        in_specs=[
            pl.BlockSpec(memory_space=pltpu.VMEM),
            pl.BlockSpec(memory_space=pltpu.VMEM),
            pl.BlockSpec(memory_space=pltpu.VMEM),
            pl.BlockSpec(memory_space=pltpu.VMEM),
            pl.BlockSpec(memory_space=pltpu.SMEM),
        ],
        out_specs=pl.BlockSpec(memory_space=pltpu.SMEM),
    )(gmin_b, gdel_b, grel_b, gscal_b, partial)


def kernel(nf1, nf2, nf3, nf4, disjoint, nf3_neg0, min_embedding,
           delta_embedding, relation_embedding, scaling_embedding):
    s1, s2, s3, s4, s5, s6 = _SAMP
    nf1_s = nf1[s1]
    nf2_s = nf2[s2]
    nf3_s = nf3[s3]
    nf4_s = nf4[s4]
    dis_s = disjoint[s5]
    neg_s = nf3_neg0[s6]

    idx_a = jnp.concatenate([
        nf1_s[:, 0], nf1_s[:, 1],
        nf2_s[:, 0], nf2_s[:, 1], nf2_s[:, 2],
        nf3_s[:, 0], nf3_s[:, 2],
        nf4_s[:, 1], nf4_s[:, 2],
        nf3_s[:, 1], nf4_s[:, 0],
    ]).astype(jnp.int32)
    idx_b = jnp.concatenate([
        dis_s[:, 0], dis_s[:, 1],
        neg_s[:, 0], neg_s[:, 2],
        neg_s[:, 1],
    ]).astype(jnp.int32)

    gmin_a, gdel_a, grel_a, gscal_a = _build_sc_gather_a()(
        min_embedding, delta_embedding, relation_embedding, scaling_embedding,
        idx_a)
    partial = _tc_loss_a(gmin_a, gdel_a, grel_a, gscal_a)
    gmin_b, gdel_b, grel_b, gscal_b = _build_sc_gather_b()(
        min_embedding, delta_embedding, relation_embedding, scaling_embedding,
        idx_b)
    res = _tc_loss_b(gmin_b, gdel_b, grel_b, gscal_b, partial)
    return res[0, 0]
